# Initial kernel scaffold; baseline (speedup 1.0000x reference)
import functools

import jax
import jax.numpy as jnp
from jax import lax
from jax.experimental import pallas as pl
from jax.experimental.pallas import tpu as pltpu
from jax.experimental.pallas import tpu_sc as plsc

E = 8
TOP_K = 2
BLK = 128
FFB = 2048


def _router_body(hs_ref, gw_ref, logits_ref, pos0_ref, pos1_ref,
                 w0_ref, w1_ref, te_ref, na_ref, s, nt):
    hs = hs_ref[...]
    gw = gw_ref[...]
    logits = lax.dot_general(hs, gw, (((1,), (1,)), ((), ())),
                             preferred_element_type=jnp.float32)
    logits_ref[...] = logits

    m = jnp.max(logits, axis=1, keepdims=True)
    ex = jnp.exp(logits - m)
    rw = ex / jnp.sum(ex, axis=1, keepdims=True)

    iota_e = lax.broadcasted_iota(jnp.int32, (s, E), 1)
    m1 = jnp.max(rw, axis=1, keepdims=True)
    idx1 = jnp.min(jnp.where(rw == m1, iota_e, E), axis=1, keepdims=True)
    oh0 = (iota_e == idx1).astype(jnp.float32)
    rw2 = jnp.where(oh0 > 0.0, -jnp.inf, rw)
    m2 = jnp.max(rw2, axis=1, keepdims=True)
    idx2 = jnp.min(jnp.where(rw2 == m2, iota_e, E), axis=1, keepdims=True)
    oh1 = (iota_e == idx2).astype(jnp.float32)

    w0 = jnp.sum(rw * oh0, axis=1, keepdims=True)
    w1 = jnp.sum(rw * oh1, axis=1, keepdims=True)
    sm = w0 + w1
    w0_ref[...] = w0 / sm
    w1_ref[...] = w1 / sm

    ir = lax.broadcasted_iota(jnp.int32, (s, s), 0)
    ic = lax.broadcasted_iota(jnp.int32, (s, s), 1)
    ltri = jnp.where(ir > ic, 1.0, 0.0).astype(jnp.bfloat16)
    oh01 = jnp.concatenate([oh0, oh1], axis=1).astype(jnp.bfloat16)
    cs = lax.dot_general(ltri, oh01, (((1,), (0,)), ((), ())),
                         preferred_element_type=jnp.float32)
    rank0 = jnp.sum(cs[:, :E] * oh0, axis=1, keepdims=True)
    rank1 = jnp.sum(cs[:, E:] * oh1, axis=1, keepdims=True)

    c0 = jnp.sum(oh0, axis=0, keepdims=True)
    c1 = jnp.sum(oh1, axis=0, keepdims=True)
    cnt = (c0 + c1).astype(jnp.int32)
    ntiles = (cnt + (BLK - 1)) >> 7
    ntiles_f = ntiles.astype(jnp.float32)
    er = lax.broadcasted_iota(jnp.int32, (E, E), 0)
    ec = lax.broadcasted_iota(jnp.int32, (E, E), 1)
    supper8 = jnp.where(er < ec, 1.0, 0.0)
    tile_off = lax.dot_general(ntiles_f, supper8, (((1,), (0,)), ((), ())),
                               preferred_element_type=jnp.float32)
    off_pad = float(BLK) * tile_off

    pos0 = jnp.sum(oh0 * off_pad, axis=1, keepdims=True) + rank0
    pos1 = jnp.sum(oh1 * (off_pad + c0), axis=1, keepdims=True) + rank1
    pos0_ref[...] = pos0.astype(jnp.int32)
    pos1_ref[...] = pos1.astype(jnp.int32)

    total = jnp.sum(ntiles, axis=1, keepdims=True)
    na_ref[...] = total
    jtile = lax.broadcasted_iota(jnp.int32, (1, nt), 1)
    jtile = jnp.minimum(jtile, total[0, 0] - 1)
    te = jnp.zeros((1, nt), jnp.int32)
    toff_i = tile_off.astype(jnp.int32)
    for e in range(E):
        lo = toff_i[0, e]
        hi = lo + ntiles[0, e]
        te = jnp.where((jtile >= lo) & (jtile < hi), e, te)
    te_ref[...] = te


def _stage_a_body(te_ref, na_ref, x_ref, w1_ref, w3_ref, g_ref):
    i = pl.program_id(1)

    @pl.when(i < na_ref[0])
    def _():
        x = x_ref[...]
        a = lax.dot_general(x, w1_ref[0], (((1,), (1,)), ((), ())),
                            preferred_element_type=jnp.float32)
        b = lax.dot_general(x, w3_ref[0], (((1,), (1,)), ((), ())),
                            preferred_element_type=jnp.float32)
        g_ref[...] = a * lax.logistic(a) * b


def _stage_b_body(te_ref, na_ref, g_ref, w2_ref, out_ref):
    i = pl.program_id(0)

    @pl.when(i < na_ref[0])
    def _():
        g = g_ref[...]
        out_ref[...] = lax.dot_general(g, w2_ref[0], (((1,), (1,)), ((), ())),
                                       preferred_element_type=jnp.float32)


def kernel(hidden_states, gate_w, w1, w2, w3):
    b, s, h = hidden_states.shape
    e, ff, _ = w1.shape
    hs = hidden_states.reshape(s, h)
    nt = (s * TOP_K) // BLK + E
    p = nt * BLK

    router = pl.pallas_call(
        functools.partial(_router_body, s=s, nt=nt),
        out_shape=(
            jax.ShapeDtypeStruct((s, e), jnp.float32),
            jax.ShapeDtypeStruct((s, 1), jnp.int32),
            jax.ShapeDtypeStruct((s, 1), jnp.int32),
            jax.ShapeDtypeStruct((s, 1), jnp.float32),
            jax.ShapeDtypeStruct((s, 1), jnp.float32),
            jax.ShapeDtypeStruct((1, nt), jnp.int32),
            jax.ShapeDtypeStruct((1, 1), jnp.int32),
        ),
    )
    logits, pos0, pos1, w0n, w1n, te, na = router(hs, gate_w)
    pos0 = pos0.reshape(s)
    pos1 = pos1.reshape(s)
    w0n = w0n.reshape(s)
    w1n = w1n.reshape(s)
    te = te.reshape(nt)
    na = na.reshape(1)

    info = plsc.get_sparse_core_info()
    nc, ns = info.num_cores, info.num_subcores
    nw = nc * ns
    tw = s // nw
    chunk = 32
    nchunk = tw // chunk

    @functools.partial(
        pl.kernel,
        out_type=jax.ShapeDtypeStruct((p, h), jnp.float32),
        mesh=plsc.VectorSubcoreMesh(core_axis_name="c", subcore_axis_name="s"),
        scratch_types=[
            pltpu.VMEM((chunk, h), jnp.float32),
            pltpu.VMEM((chunk,), jnp.int32),
            pltpu.VMEM((chunk,), jnp.int32),
            pltpu.SemaphoreType.DMA,
            pltpu.SemaphoreType.DMA,
        ],
    )
    def dispatch(hs_hbm, p0_hbm, p1_hbm, xs_hbm, rows_v, p0_v, p1_v, s0, s1):
        wid = lax.axis_index("s") * nc + lax.axis_index("c")
        for k in range(nchunk):
            base = wid * tw + k * chunk
            pltpu.sync_copy(hs_hbm.at[pl.ds(base, chunk)], rows_v)
            pltpu.sync_copy(p0_hbm.at[pl.ds(base, chunk)], p0_v)
            pltpu.sync_copy(p1_hbm.at[pl.ds(base, chunk)], p1_v)
            c0 = pltpu.make_async_copy(rows_v, xs_hbm.at[p0_v], s0)
            c1 = pltpu.make_async_copy(rows_v, xs_hbm.at[p1_v], s1)
            c0.start()
            c1.start()
            c0.wait()
            c1.wait()

    x_sorted = dispatch(hs, pos0, pos1)

    nff = ff // FFB
    stage_a = pl.pallas_call(
        _stage_a_body,
        grid_spec=pltpu.PrefetchScalarGridSpec(
            num_scalar_prefetch=2,
            grid=(nff, nt),
            in_specs=[
                pl.BlockSpec((BLK, h),
                             lambda fj, i, te, na: (jnp.minimum(i, na[0] - 1), 0)),
                pl.BlockSpec((1, FFB, h),
                             lambda fj, i, te, na: (te[jnp.minimum(i, na[0] - 1)], fj, 0)),
                pl.BlockSpec((1, FFB, h),
                             lambda fj, i, te, na: (te[jnp.minimum(i, na[0] - 1)], fj, 0)),
            ],
            out_specs=pl.BlockSpec((BLK, FFB), lambda fj, i, te, na: (i, fj)),
        ),
        out_shape=jax.ShapeDtypeStruct((p, ff), jnp.float32),
        compiler_params=pltpu.CompilerParams(
            dimension_semantics=("arbitrary", "arbitrary")),
    )
    g = stage_a(te, na, x_sorted, w1, w3)

    stage_b = pl.pallas_call(
        _stage_b_body,
        grid_spec=pltpu.PrefetchScalarGridSpec(
            num_scalar_prefetch=2,
            grid=(nt,),
            in_specs=[
                pl.BlockSpec((BLK, ff),
                             lambda i, te, na: (jnp.minimum(i, na[0] - 1), 0)),
                pl.BlockSpec((1, h, ff),
                             lambda i, te, na: (te[jnp.minimum(i, na[0] - 1)], 0, 0)),
            ],
            out_specs=pl.BlockSpec((BLK, h), lambda i, te, na: (i, 0)),
        ),
        out_shape=jax.ShapeDtypeStruct((p, h), jnp.float32),
        compiler_params=pltpu.CompilerParams(
            dimension_semantics=("arbitrary",)),
    )
    out_sorted = stage_b(te, na, g, w2)

    c2 = 16
    nchunk2 = tw // c2
    hv = h // 16

    @functools.partial(
        pl.kernel,
        out_type=jax.ShapeDtypeStruct((s, h), jnp.float32),
        mesh=plsc.VectorSubcoreMesh(core_axis_name="c", subcore_axis_name="s"),
        scratch_types=[
            pltpu.VMEM((c2, h), jnp.float32),
            pltpu.VMEM((c2, h), jnp.float32),
            pltpu.VMEM((c2, h), jnp.float32),
            pltpu.VMEM((c2,), jnp.int32),
            pltpu.VMEM((c2,), jnp.int32),
            pltpu.VMEM((c2,), jnp.float32),
            pltpu.VMEM((c2,), jnp.float32),
            pltpu.SemaphoreType.DMA,
            pltpu.SemaphoreType.DMA,
        ],
    )
    def combine(os_hbm, p0_hbm, p1_hbm, w0_hbm, w1_hbm, fin_hbm,
                a_v, b_v, o_v, p0_v, p1_v, w0_v, w1_v, s0, s1):
        wid = lax.axis_index("s") * nc + lax.axis_index("c")
        for k in range(nchunk2):
            base = wid * tw + k * c2
            pltpu.sync_copy(p0_hbm.at[pl.ds(base, c2)], p0_v)
            pltpu.sync_copy(p1_hbm.at[pl.ds(base, c2)], p1_v)
            pltpu.sync_copy(w0_hbm.at[pl.ds(base, c2)], w0_v)
            pltpu.sync_copy(w1_hbm.at[pl.ds(base, c2)], w1_v)
            ca = pltpu.make_async_copy(os_hbm.at[p0_v], a_v, s0)
            cb = pltpu.make_async_copy(os_hbm.at[p1_v], b_v, s1)
            ca.start()
            cb.start()
            ca.wait()
            cb.wait()
            for j in range(c2):
                jvec = jnp.full((16,), j, jnp.int32)
                w0s = plsc.load_gather(w0_v, [jvec])
                w1s = plsc.load_gather(w1_v, [jvec])

                def hbody(hh, carry):
                    for u in range(4):
                        sl = pl.ds((hh * 4 + u) * 16, 16)
                        o_v[j, sl] = w0s * a_v[j, sl] + w1s * b_v[j, sl]
                    return carry

                lax.fori_loop(0, hv // 4, hbody, 0)
            pltpu.sync_copy(o_v, fin_hbm.at[pl.ds(base, c2)])

    final = combine(out_sorted, pos0, pos1, w0n, w1n)
    return final.reshape(b, s, h), logits


# speedup vs baseline: 1.2491x; 1.2491x over previous
import functools

import jax
import jax.numpy as jnp
from jax import lax
from jax.experimental import pallas as pl
from jax.experimental.pallas import tpu as pltpu
from jax.experimental.pallas import tpu_sc as plsc

E = 8
TOP_K = 2
BLK = 128
FFB = 1024
HB = 1024


def _router_body(hs_ref, gw_ref, logits_ref, pos0_ref, pos1_ref,
                 w0_ref, w1_ref, te_ref, na_ref, s, nt):
    hs = hs_ref[...]
    gw = gw_ref[...]
    logits = lax.dot_general(hs, gw, (((1,), (1,)), ((), ())),
                             preferred_element_type=jnp.float32)
    logits_ref[...] = logits

    m = jnp.max(logits, axis=1, keepdims=True)
    ex = jnp.exp(logits - m)
    rw = ex / jnp.sum(ex, axis=1, keepdims=True)

    iota_e = lax.broadcasted_iota(jnp.int32, (s, E), 1)
    m1 = jnp.max(rw, axis=1, keepdims=True)
    idx1 = jnp.min(jnp.where(rw == m1, iota_e, E), axis=1, keepdims=True)
    oh0 = (iota_e == idx1).astype(jnp.float32)
    rw2 = jnp.where(oh0 > 0.0, -jnp.inf, rw)
    m2 = jnp.max(rw2, axis=1, keepdims=True)
    idx2 = jnp.min(jnp.where(rw2 == m2, iota_e, E), axis=1, keepdims=True)
    oh1 = (iota_e == idx2).astype(jnp.float32)

    w0 = jnp.sum(rw * oh0, axis=1, keepdims=True)
    w1 = jnp.sum(rw * oh1, axis=1, keepdims=True)
    sm = w0 + w1
    w0_ref[...] = w0 / sm
    w1_ref[...] = w1 / sm

    ir = lax.broadcasted_iota(jnp.int32, (s, s), 0)
    ic = lax.broadcasted_iota(jnp.int32, (s, s), 1)
    ltri = jnp.where(ir > ic, 1.0, 0.0).astype(jnp.bfloat16)
    oh01 = jnp.concatenate([oh0, oh1], axis=1).astype(jnp.bfloat16)
    cs = lax.dot_general(ltri, oh01, (((1,), (0,)), ((), ())),
                         preferred_element_type=jnp.float32)
    rank0 = jnp.sum(cs[:, :E] * oh0, axis=1, keepdims=True)
    rank1 = jnp.sum(cs[:, E:] * oh1, axis=1, keepdims=True)

    c0 = jnp.sum(oh0, axis=0, keepdims=True)
    c1 = jnp.sum(oh1, axis=0, keepdims=True)
    cnt = (c0 + c1).astype(jnp.int32)
    ntiles = (cnt + (BLK - 1)) >> 7
    ntiles_f = ntiles.astype(jnp.float32)
    er = lax.broadcasted_iota(jnp.int32, (E, E), 0)
    ec = lax.broadcasted_iota(jnp.int32, (E, E), 1)
    supper8 = jnp.where(er < ec, 1.0, 0.0)
    tile_off = lax.dot_general(ntiles_f, supper8, (((1,), (0,)), ((), ())),
                               preferred_element_type=jnp.float32)
    off_pad = float(BLK) * tile_off

    pos0 = jnp.sum(oh0 * off_pad, axis=1, keepdims=True) + rank0
    pos1 = jnp.sum(oh1 * (off_pad + c0), axis=1, keepdims=True) + rank1
    pos0_ref[...] = pos0.astype(jnp.int32)
    pos1_ref[...] = pos1.astype(jnp.int32)

    total = jnp.sum(ntiles, axis=1, keepdims=True)
    na_ref[...] = total
    jtile = lax.broadcasted_iota(jnp.int32, (1, nt), 1)
    jtile = jnp.minimum(jtile, total[0, 0] - 1)
    te = jnp.zeros((1, nt), jnp.int32)
    toff_i = tile_off.astype(jnp.int32)
    for e in range(E):
        lo = toff_i[0, e]
        hi = lo + ntiles[0, e]
        te = jnp.where((jtile >= lo) & (jtile < hi), e, te)
    te_ref[...] = te


def _stage_a_body(te_ref, na_ref, x_ref, w1_ref, w3_ref, g_ref):
    i = pl.program_id(1)

    @pl.when(i < na_ref[0])
    def _():
        x = x_ref[...]
        a = lax.dot_general(x, w1_ref[0], (((1,), (1,)), ((), ())),
                            preferred_element_type=jnp.float32)
        b = lax.dot_general(x, w3_ref[0], (((1,), (1,)), ((), ())),
                            preferred_element_type=jnp.float32)
        g_ref[...] = a * lax.logistic(a) * b


def _stage_b_body(te_ref, na_ref, g_ref, w2_ref, out_ref):
    i = pl.program_id(1)

    @pl.when(i < na_ref[0])
    def _():
        g = g_ref[...]
        out_ref[...] = lax.dot_general(g, w2_ref[0], (((1,), (1,)), ((), ())),
                                       preferred_element_type=jnp.float32)


def kernel(hidden_states, gate_w, w1, w2, w3):
    b, s, h = hidden_states.shape
    e, ff, _ = w1.shape
    hs = hidden_states.reshape(s, h)
    nt = (s * TOP_K) // BLK + E
    p = nt * BLK

    router = pl.pallas_call(
        functools.partial(_router_body, s=s, nt=nt),
        out_shape=(
            jax.ShapeDtypeStruct((s, e), jnp.float32),
            jax.ShapeDtypeStruct((s, 1), jnp.int32),
            jax.ShapeDtypeStruct((s, 1), jnp.int32),
            jax.ShapeDtypeStruct((s, 1), jnp.float32),
            jax.ShapeDtypeStruct((s, 1), jnp.float32),
            jax.ShapeDtypeStruct((1, nt), jnp.int32),
            jax.ShapeDtypeStruct((1, 1), jnp.int32),
        ),
    )
    logits, pos0, pos1, w0n, w1n, te, na = router(hs, gate_w)
    pos0 = pos0.reshape(s)
    pos1 = pos1.reshape(s)
    w0n = w0n.reshape(s)
    w1n = w1n.reshape(s)
    te = te.reshape(nt)
    na = na.reshape(1)

    nc, ns = 2, 16
    nw = nc * ns
    tw = s // nw
    chunk = 32
    nchunk = tw // chunk

    @functools.partial(
        pl.kernel,
        out_type=jax.ShapeDtypeStruct((p, h), jnp.float32),
        mesh=plsc.VectorSubcoreMesh(core_axis_name="c", subcore_axis_name="s"),
        scratch_types=[
            pltpu.VMEM((chunk, h), jnp.float32),
            pltpu.VMEM((chunk,), jnp.int32),
            pltpu.VMEM((chunk,), jnp.int32),
            pltpu.SemaphoreType.DMA,
            pltpu.SemaphoreType.DMA,
        ],
    )
    def dispatch(hs_hbm, p0_hbm, p1_hbm, xs_hbm, rows_v, p0_v, p1_v, s0, s1):
        wid = lax.axis_index("s") * nc + lax.axis_index("c")
        for k in range(nchunk):
            base = wid * tw + k * chunk
            pltpu.sync_copy(hs_hbm.at[pl.ds(base, chunk)], rows_v)
            pltpu.sync_copy(p0_hbm.at[pl.ds(base, chunk)], p0_v)
            pltpu.sync_copy(p1_hbm.at[pl.ds(base, chunk)], p1_v)
            c0 = pltpu.make_async_copy(rows_v, xs_hbm.at[p0_v], s0)
            c1 = pltpu.make_async_copy(rows_v, xs_hbm.at[p1_v], s1)
            c0.start()
            c1.start()
            c0.wait()
            c1.wait()

    x_sorted = dispatch(hs, pos0, pos1)

    nff = ff // FFB
    stage_a = pl.pallas_call(
        _stage_a_body,
        grid_spec=pltpu.PrefetchScalarGridSpec(
            num_scalar_prefetch=2,
            grid=(nff, nt),
            in_specs=[
                pl.BlockSpec((BLK, h),
                             lambda fj, i, te, na: (jnp.minimum(i, na[0] - 1), 0)),
                pl.BlockSpec((1, FFB, h),
                             lambda fj, i, te, na: (te[jnp.minimum(i, na[0] - 1)], fj, 0)),
                pl.BlockSpec((1, FFB, h),
                             lambda fj, i, te, na: (te[jnp.minimum(i, na[0] - 1)], fj, 0)),
            ],
            out_specs=pl.BlockSpec((BLK, FFB), lambda fj, i, te, na: (i, fj)),
        ),
        out_shape=jax.ShapeDtypeStruct((p, ff), jnp.float32),
        compiler_params=pltpu.CompilerParams(
            dimension_semantics=("arbitrary", "arbitrary")),
    )
    g = stage_a(te, na, x_sorted, w1, w3)

    stage_b = pl.pallas_call(
        _stage_b_body,
        grid_spec=pltpu.PrefetchScalarGridSpec(
            num_scalar_prefetch=2,
            grid=(h // HB, nt),
            in_specs=[
                pl.BlockSpec((BLK, ff),
                             lambda hb, i, te, na: (jnp.minimum(i, na[0] - 1), 0)),
                pl.BlockSpec((1, HB, ff),
                             lambda hb, i, te, na: (te[jnp.minimum(i, na[0] - 1)], hb, 0)),
            ],
            out_specs=pl.BlockSpec((BLK, HB), lambda hb, i, te, na: (i, hb)),
        ),
        out_shape=jax.ShapeDtypeStruct((p, h), jnp.float32),
        compiler_params=pltpu.CompilerParams(
            dimension_semantics=("arbitrary", "arbitrary")),
    )
    out_sorted = stage_b(te, na, g, w2)

    c2 = 16
    nchunk2 = tw // c2
    hv = h // 16

    @functools.partial(
        pl.kernel,
        out_type=jax.ShapeDtypeStruct((s, h), jnp.float32),
        mesh=plsc.VectorSubcoreMesh(core_axis_name="c", subcore_axis_name="s"),
        scratch_types=[
            pltpu.VMEM((c2, h), jnp.float32),
            pltpu.VMEM((c2, h), jnp.float32),
            pltpu.VMEM((c2, h), jnp.float32),
            pltpu.VMEM((c2,), jnp.int32),
            pltpu.VMEM((c2,), jnp.int32),
            pltpu.VMEM((c2,), jnp.float32),
            pltpu.VMEM((c2,), jnp.float32),
            pltpu.SemaphoreType.DMA,
            pltpu.SemaphoreType.DMA,
        ],
    )
    def combine(os_hbm, p0_hbm, p1_hbm, w0_hbm, w1_hbm, fin_hbm,
                a_v, b_v, o_v, p0_v, p1_v, w0_v, w1_v, s0, s1):
        wid = lax.axis_index("s") * nc + lax.axis_index("c")
        for k in range(nchunk2):
            base = wid * tw + k * c2
            pltpu.sync_copy(p0_hbm.at[pl.ds(base, c2)], p0_v)
            pltpu.sync_copy(p1_hbm.at[pl.ds(base, c2)], p1_v)
            pltpu.sync_copy(w0_hbm.at[pl.ds(base, c2)], w0_v)
            pltpu.sync_copy(w1_hbm.at[pl.ds(base, c2)], w1_v)
            ca = pltpu.make_async_copy(os_hbm.at[p0_v], a_v, s0)
            cb = pltpu.make_async_copy(os_hbm.at[p1_v], b_v, s1)
            ca.start()
            cb.start()
            ca.wait()
            cb.wait()
            w0vec = w0_v[...]
            w1vec = w1_v[...]
            gdn = lax.GatherDimensionNumbers(
                offset_dims=(), collapsed_slice_dims=(0,), start_index_map=(0,))
            for j in range(c2):
                jvec = jnp.full((c2, 1), j, jnp.int32)
                w0s = lax.gather(w0vec, jvec, gdn, (1,),
                                 mode=lax.GatherScatterMode.PROMISE_IN_BOUNDS)
                w1s = lax.gather(w1vec, jvec, gdn, (1,),
                                 mode=lax.GatherScatterMode.PROMISE_IN_BOUNDS)

                def hbody(hh, carry):
                    for u in range(4):
                        sl = pl.ds((hh * 4 + u) * 16, 16)
                        o_v[j, sl] = w0s * a_v[j, sl] + w1s * b_v[j, sl]
                    return carry

                lax.fori_loop(0, hv // 4, hbody, 0)
            pltpu.sync_copy(o_v, fin_hbm.at[pl.ds(base, c2)])

    final = combine(out_sorted, pos0, pos1, w0n, w1n)
    return final.reshape(b, s, h), logits


# speedup vs baseline: 1.7882x; 1.4316x over previous
import functools

import jax
import jax.numpy as jnp
from jax import lax
from jax.experimental import pallas as pl
from jax.experimental.pallas import tpu as pltpu
from jax.experimental.pallas import tpu_sc as plsc

E = 8
TOP_K = 2
BLK = 256
FFB = 1024
HB = 1024


def _router_body(hs_ref, gw_ref, logits_ref, pos0_ref, pos1_ref,
                 w0_ref, w1_ref, te_ref, na_ref, s, nt):
    hs = hs_ref[...]
    gw = gw_ref[...]
    logits = lax.dot_general(hs, gw, (((1,), (1,)), ((), ())),
                             preferred_element_type=jnp.float32)
    logits_ref[...] = logits

    m = jnp.max(logits, axis=1, keepdims=True)
    ex = jnp.exp(logits - m)
    rw = ex / jnp.sum(ex, axis=1, keepdims=True)

    iota_e = lax.broadcasted_iota(jnp.int32, (s, E), 1)
    m1 = jnp.max(rw, axis=1, keepdims=True)
    idx1 = jnp.min(jnp.where(rw == m1, iota_e, E), axis=1, keepdims=True)
    oh0 = (iota_e == idx1).astype(jnp.float32)
    rw2 = jnp.where(oh0 > 0.0, -jnp.inf, rw)
    m2 = jnp.max(rw2, axis=1, keepdims=True)
    idx2 = jnp.min(jnp.where(rw2 == m2, iota_e, E), axis=1, keepdims=True)
    oh1 = (iota_e == idx2).astype(jnp.float32)

    w0 = jnp.sum(rw * oh0, axis=1, keepdims=True)
    w1 = jnp.sum(rw * oh1, axis=1, keepdims=True)
    sm = w0 + w1
    w0_ref[...] = w0 / sm
    w1_ref[...] = w1 / sm

    ir = lax.broadcasted_iota(jnp.int32, (s, s), 0)
    ic = lax.broadcasted_iota(jnp.int32, (s, s), 1)
    ltri = jnp.where(ir > ic, 1.0, 0.0).astype(jnp.bfloat16)
    oh01 = jnp.concatenate([oh0, oh1], axis=1).astype(jnp.bfloat16)
    cs = lax.dot_general(ltri, oh01, (((1,), (0,)), ((), ())),
                         preferred_element_type=jnp.float32)
    rank0 = jnp.sum(cs[:, :E] * oh0, axis=1, keepdims=True)
    rank1 = jnp.sum(cs[:, E:] * oh1, axis=1, keepdims=True)

    c0 = jnp.sum(oh0, axis=0, keepdims=True)
    c1 = jnp.sum(oh1, axis=0, keepdims=True)
    cnt = (c0 + c1).astype(jnp.int32)
    ntiles = (cnt + (BLK - 1)) >> BLK.bit_length() - 1
    ntiles_f = ntiles.astype(jnp.float32)
    er = lax.broadcasted_iota(jnp.int32, (E, E), 0)
    ec = lax.broadcasted_iota(jnp.int32, (E, E), 1)
    supper8 = jnp.where(er < ec, 1.0, 0.0)
    tile_off = lax.dot_general(ntiles_f, supper8, (((1,), (0,)), ((), ())),
                               preferred_element_type=jnp.float32)
    off_pad = float(BLK) * tile_off

    pos0 = jnp.sum(oh0 * off_pad, axis=1, keepdims=True) + rank0
    pos1 = jnp.sum(oh1 * (off_pad + c0), axis=1, keepdims=True) + rank1
    pos0_ref[...] = pos0.astype(jnp.int32)
    pos1_ref[...] = pos1.astype(jnp.int32)

    total = jnp.sum(ntiles, axis=1, keepdims=True)
    na_ref[...] = total
    jtile = lax.broadcasted_iota(jnp.int32, (1, nt), 1)
    jtile = jnp.minimum(jtile, total[0, 0] - 1)
    te = jnp.zeros((1, nt), jnp.int32)
    toff_i = tile_off.astype(jnp.int32)
    for e in range(E):
        lo = toff_i[0, e]
        hi = lo + ntiles[0, e]
        te = jnp.where((jtile >= lo) & (jtile < hi), e, te)
    te_ref[...] = te


def _stage_a_body(te_ref, na_ref, x_ref, w1_ref, w3_ref, g_ref):
    i = pl.program_id(1)

    @pl.when(i < na_ref[0])
    def _():
        x = x_ref[...].astype(jnp.bfloat16)
        a = lax.dot_general(x, w1_ref[0].astype(jnp.bfloat16),
                            (((1,), (1,)), ((), ())),
                            preferred_element_type=jnp.float32)
        b = lax.dot_general(x, w3_ref[0].astype(jnp.bfloat16),
                            (((1,), (1,)), ((), ())),
                            preferred_element_type=jnp.float32)
        g_ref[...] = a * lax.logistic(a) * b


def _stage_b_body(te_ref, na_ref, g_ref, w2_ref, out_ref):
    i = pl.program_id(1)

    @pl.when(i < na_ref[0])
    def _():
        g = g_ref[...].astype(jnp.bfloat16)
        out_ref[...] = lax.dot_general(g, w2_ref[0].astype(jnp.bfloat16),
                                       (((1,), (1,)), ((), ())),
                                       preferred_element_type=jnp.float32)


def kernel(hidden_states, gate_w, w1, w2, w3):
    b, s, h = hidden_states.shape
    e, ff, _ = w1.shape
    hs = hidden_states.reshape(s, h)
    nt = (s * TOP_K) // BLK + E
    p = nt * BLK

    router = pl.pallas_call(
        functools.partial(_router_body, s=s, nt=nt),
        out_shape=(
            jax.ShapeDtypeStruct((s, e), jnp.float32),
            jax.ShapeDtypeStruct((s, 1), jnp.int32),
            jax.ShapeDtypeStruct((s, 1), jnp.int32),
            jax.ShapeDtypeStruct((s, 1), jnp.float32),
            jax.ShapeDtypeStruct((s, 1), jnp.float32),
            jax.ShapeDtypeStruct((1, nt), jnp.int32),
            jax.ShapeDtypeStruct((1, 1), jnp.int32),
        ),
    )
    logits, pos0, pos1, w0n, w1n, te, na = router(hs, gate_w)
    pos0 = pos0.reshape(s)
    pos1 = pos1.reshape(s)
    w0n = w0n.reshape(s)
    w1n = w1n.reshape(s)
    te = te.reshape(nt)
    na = na.reshape(1)

    nc, ns = 2, 16
    nw = nc * ns
    tw = s // nw
    chunk = 32
    nchunk = tw // chunk

    @functools.partial(
        pl.kernel,
        out_type=jax.ShapeDtypeStruct((p, h), jnp.float32),
        mesh=plsc.VectorSubcoreMesh(core_axis_name="c", subcore_axis_name="s"),
        scratch_types=[
            pltpu.VMEM((chunk, h), jnp.float32),
            pltpu.VMEM((chunk,), jnp.int32),
            pltpu.VMEM((chunk,), jnp.int32),
            pltpu.SemaphoreType.DMA,
            pltpu.SemaphoreType.DMA,
        ],
    )
    def dispatch(hs_hbm, p0_hbm, p1_hbm, xs_hbm, rows_v, p0_v, p1_v, s0, s1):
        wid = lax.axis_index("s") * nc + lax.axis_index("c")
        for k in range(nchunk):
            base = wid * tw + k * chunk
            pltpu.sync_copy(hs_hbm.at[pl.ds(base, chunk)], rows_v)
            pltpu.sync_copy(p0_hbm.at[pl.ds(base, chunk)], p0_v)
            pltpu.sync_copy(p1_hbm.at[pl.ds(base, chunk)], p1_v)
            c0 = pltpu.make_async_copy(rows_v, xs_hbm.at[p0_v], s0)
            c1 = pltpu.make_async_copy(rows_v, xs_hbm.at[p1_v], s1)
            c0.start()
            c1.start()
            c0.wait()
            c1.wait()

    x_sorted = dispatch(hs, pos0, pos1)

    nff = ff // FFB
    stage_a = pl.pallas_call(
        _stage_a_body,
        grid_spec=pltpu.PrefetchScalarGridSpec(
            num_scalar_prefetch=2,
            grid=(nff, nt),
            in_specs=[
                pl.BlockSpec((BLK, h),
                             lambda fj, i, te, na: (jnp.minimum(i, na[0] - 1), 0)),
                pl.BlockSpec((1, FFB, h),
                             lambda fj, i, te, na: (te[jnp.minimum(i, na[0] - 1)], fj, 0)),
                pl.BlockSpec((1, FFB, h),
                             lambda fj, i, te, na: (te[jnp.minimum(i, na[0] - 1)], fj, 0)),
            ],
            out_specs=pl.BlockSpec((BLK, FFB), lambda fj, i, te, na: (i, fj)),
        ),
        out_shape=jax.ShapeDtypeStruct((p, ff), jnp.float32),
        compiler_params=pltpu.CompilerParams(
            dimension_semantics=("arbitrary", "arbitrary")),
    )
    g = stage_a(te, na, x_sorted, w1, w3)

    stage_b = pl.pallas_call(
        _stage_b_body,
        grid_spec=pltpu.PrefetchScalarGridSpec(
            num_scalar_prefetch=2,
            grid=(h // HB, nt),
            in_specs=[
                pl.BlockSpec((BLK, ff),
                             lambda hb, i, te, na: (jnp.minimum(i, na[0] - 1), 0)),
                pl.BlockSpec((1, HB, ff),
                             lambda hb, i, te, na: (te[jnp.minimum(i, na[0] - 1)], hb, 0)),
            ],
            out_specs=pl.BlockSpec((BLK, HB), lambda hb, i, te, na: (i, hb)),
        ),
        out_shape=jax.ShapeDtypeStruct((p, h), jnp.float32),
        compiler_params=pltpu.CompilerParams(
            dimension_semantics=("arbitrary", "arbitrary")),
    )
    out_sorted = stage_b(te, na, g, w2)

    c2 = 16
    nchunk2 = tw // c2
    hv = h // 16

    @functools.partial(
        pl.kernel,
        out_type=jax.ShapeDtypeStruct((s, h), jnp.float32),
        mesh=plsc.VectorSubcoreMesh(core_axis_name="c", subcore_axis_name="s"),
        scratch_types=[
            pltpu.VMEM((c2, h), jnp.float32),
            pltpu.VMEM((c2, h), jnp.float32),
            pltpu.VMEM((c2, h), jnp.float32),
            pltpu.VMEM((c2,), jnp.int32),
            pltpu.VMEM((c2,), jnp.int32),
            pltpu.VMEM((c2,), jnp.float32),
            pltpu.VMEM((c2,), jnp.float32),
            pltpu.SemaphoreType.DMA,
            pltpu.SemaphoreType.DMA,
        ],
    )
    def combine(os_hbm, p0_hbm, p1_hbm, w0_hbm, w1_hbm, fin_hbm,
                a_v, b_v, o_v, p0_v, p1_v, w0_v, w1_v, s0, s1):
        wid = lax.axis_index("s") * nc + lax.axis_index("c")
        for k in range(nchunk2):
            base = wid * tw + k * c2
            pltpu.sync_copy(p0_hbm.at[pl.ds(base, c2)], p0_v)
            pltpu.sync_copy(p1_hbm.at[pl.ds(base, c2)], p1_v)
            pltpu.sync_copy(w0_hbm.at[pl.ds(base, c2)], w0_v)
            pltpu.sync_copy(w1_hbm.at[pl.ds(base, c2)], w1_v)
            ca = pltpu.make_async_copy(os_hbm.at[p0_v], a_v, s0)
            cb = pltpu.make_async_copy(os_hbm.at[p1_v], b_v, s1)
            ca.start()
            cb.start()
            ca.wait()
            cb.wait()
            w0vec = w0_v[...]
            w1vec = w1_v[...]
            gdn = lax.GatherDimensionNumbers(
                offset_dims=(), collapsed_slice_dims=(0,), start_index_map=(0,))
            for j in range(c2):
                jvec = jnp.full((c2, 1), j, jnp.int32)
                w0s = lax.gather(w0vec, jvec, gdn, (1,),
                                 mode=lax.GatherScatterMode.PROMISE_IN_BOUNDS)
                w1s = lax.gather(w1vec, jvec, gdn, (1,),
                                 mode=lax.GatherScatterMode.PROMISE_IN_BOUNDS)

                def hbody(hh, carry):
                    for u in range(4):
                        sl = pl.ds((hh * 4 + u) * 16, 16)
                        o_v[j, sl] = w0s * a_v[j, sl] + w1s * b_v[j, sl]
                    return carry

                lax.fori_loop(0, hv // 4, hbody, 0)
            pltpu.sync_copy(o_v, fin_hbm.at[pl.ds(base, c2)])

    final = combine(out_sorted, pos0, pos1, w0n, w1n)
    return final.reshape(b, s, h), logits


# speedup vs baseline: 1.8025x; 1.0080x over previous
import functools

import jax
import jax.numpy as jnp
from jax import lax
from jax.experimental import pallas as pl
from jax.experimental.pallas import tpu as pltpu
from jax.experimental.pallas import tpu_sc as plsc

E = 8
TOP_K = 2
BLK = 512
FFB = 1024
HB = 1024


def _router_body(hs_ref, gw_ref, logits_ref, pos0_ref, pos1_ref,
                 w0_ref, w1_ref, te_ref, na_ref, s, nt):
    hs = hs_ref[...]
    gw = gw_ref[...]
    logits = lax.dot_general(hs, gw, (((1,), (1,)), ((), ())),
                             preferred_element_type=jnp.float32)
    logits_ref[...] = logits

    m = jnp.max(logits, axis=1, keepdims=True)
    ex = jnp.exp(logits - m)
    rw = ex / jnp.sum(ex, axis=1, keepdims=True)

    iota_e = lax.broadcasted_iota(jnp.int32, (s, E), 1)
    m1 = jnp.max(rw, axis=1, keepdims=True)
    idx1 = jnp.min(jnp.where(rw == m1, iota_e, E), axis=1, keepdims=True)
    oh0 = (iota_e == idx1).astype(jnp.float32)
    rw2 = jnp.where(oh0 > 0.0, -jnp.inf, rw)
    m2 = jnp.max(rw2, axis=1, keepdims=True)
    idx2 = jnp.min(jnp.where(rw2 == m2, iota_e, E), axis=1, keepdims=True)
    oh1 = (iota_e == idx2).astype(jnp.float32)

    w0 = jnp.sum(rw * oh0, axis=1, keepdims=True)
    w1 = jnp.sum(rw * oh1, axis=1, keepdims=True)
    sm = w0 + w1
    w0_ref[...] = w0 / sm
    w1_ref[...] = w1 / sm

    ir = lax.broadcasted_iota(jnp.int32, (s, s), 0)
    ic = lax.broadcasted_iota(jnp.int32, (s, s), 1)
    ltri = jnp.where(ir > ic, 1.0, 0.0).astype(jnp.bfloat16)
    oh01 = jnp.concatenate([oh0, oh1], axis=1).astype(jnp.bfloat16)
    cs = lax.dot_general(ltri, oh01, (((1,), (0,)), ((), ())),
                         preferred_element_type=jnp.float32)
    rank0 = jnp.sum(cs[:, :E] * oh0, axis=1, keepdims=True)
    rank1 = jnp.sum(cs[:, E:] * oh1, axis=1, keepdims=True)

    c0 = jnp.sum(oh0, axis=0, keepdims=True)
    c1 = jnp.sum(oh1, axis=0, keepdims=True)
    cnt = (c0 + c1).astype(jnp.int32)
    ntiles = (cnt + (BLK - 1)) >> BLK.bit_length() - 1
    ntiles_f = ntiles.astype(jnp.float32)
    er = lax.broadcasted_iota(jnp.int32, (E, E), 0)
    ec = lax.broadcasted_iota(jnp.int32, (E, E), 1)
    supper8 = jnp.where(er < ec, 1.0, 0.0)
    tile_off = lax.dot_general(ntiles_f, supper8, (((1,), (0,)), ((), ())),
                               preferred_element_type=jnp.float32)
    off_pad = float(BLK) * tile_off

    pos0 = jnp.sum(oh0 * off_pad, axis=1, keepdims=True) + rank0
    pos1 = jnp.sum(oh1 * (off_pad + c0), axis=1, keepdims=True) + rank1
    pos0_ref[...] = pos0.astype(jnp.int32)
    pos1_ref[...] = pos1.astype(jnp.int32)

    total = jnp.sum(ntiles, axis=1, keepdims=True)
    na_ref[...] = total
    jtile = lax.broadcasted_iota(jnp.int32, (1, nt), 1)
    jtile = jnp.minimum(jtile, total[0, 0] - 1)
    te = jnp.zeros((1, nt), jnp.int32)
    toff_i = tile_off.astype(jnp.int32)
    for e in range(E):
        lo = toff_i[0, e]
        hi = lo + ntiles[0, e]
        te = jnp.where((jtile >= lo) & (jtile < hi), e, te)
    te_ref[...] = te


def _stage_a_body(te_ref, na_ref, x_ref, w1_ref, w3_ref, g_ref):
    i = pl.program_id(1)

    @pl.when(i < na_ref[0])
    def _():
        x = x_ref[...].astype(jnp.bfloat16)
        a = lax.dot_general(x, w1_ref[0].astype(jnp.bfloat16),
                            (((1,), (1,)), ((), ())),
                            preferred_element_type=jnp.float32)
        b = lax.dot_general(x, w3_ref[0].astype(jnp.bfloat16),
                            (((1,), (1,)), ((), ())),
                            preferred_element_type=jnp.float32)
        g_ref[...] = a * lax.logistic(a) * b


def _stage_b_body(te_ref, na_ref, g_ref, w2_ref, out_ref):
    i = pl.program_id(1)

    @pl.when(i < na_ref[0])
    def _():
        g = g_ref[...].astype(jnp.bfloat16)
        out_ref[...] = lax.dot_general(g, w2_ref[0].astype(jnp.bfloat16),
                                       (((1,), (1,)), ((), ())),
                                       preferred_element_type=jnp.float32)


def kernel(hidden_states, gate_w, w1, w2, w3):
    b, s, h = hidden_states.shape
    e, ff, _ = w1.shape
    hs = hidden_states.reshape(s, h)
    nt = (s * TOP_K) // BLK + E
    p = nt * BLK

    router = pl.pallas_call(
        functools.partial(_router_body, s=s, nt=nt),
        out_shape=(
            jax.ShapeDtypeStruct((s, e), jnp.float32),
            jax.ShapeDtypeStruct((s, 1), jnp.int32),
            jax.ShapeDtypeStruct((s, 1), jnp.int32),
            jax.ShapeDtypeStruct((s, 1), jnp.float32),
            jax.ShapeDtypeStruct((s, 1), jnp.float32),
            jax.ShapeDtypeStruct((1, nt), jnp.int32),
            jax.ShapeDtypeStruct((1, 1), jnp.int32),
        ),
    )
    logits, pos0, pos1, w0n, w1n, te, na = router(hs, gate_w)
    pos0 = pos0.reshape(s)
    pos1 = pos1.reshape(s)
    w0n = w0n.reshape(s)
    w1n = w1n.reshape(s)
    te = te.reshape(nt)
    na = na.reshape(1)

    nc, ns = 2, 16
    nw = nc * ns
    tw = s // nw
    chunk = 32
    nchunk = tw // chunk

    @functools.partial(
        pl.kernel,
        out_type=jax.ShapeDtypeStruct((p, h), jnp.float32),
        mesh=plsc.VectorSubcoreMesh(core_axis_name="c", subcore_axis_name="s"),
        scratch_types=[
            pltpu.VMEM((chunk, h), jnp.float32),
            pltpu.VMEM((chunk,), jnp.int32),
            pltpu.VMEM((chunk,), jnp.int32),
            pltpu.SemaphoreType.DMA,
            pltpu.SemaphoreType.DMA,
        ],
    )
    def dispatch(hs_hbm, p0_hbm, p1_hbm, xs_hbm, rows_v, p0_v, p1_v, s0, s1):
        wid = lax.axis_index("s") * nc + lax.axis_index("c")
        for k in range(nchunk):
            base = wid * tw + k * chunk
            pltpu.sync_copy(hs_hbm.at[pl.ds(base, chunk)], rows_v)
            pltpu.sync_copy(p0_hbm.at[pl.ds(base, chunk)], p0_v)
            pltpu.sync_copy(p1_hbm.at[pl.ds(base, chunk)], p1_v)
            c0 = pltpu.make_async_copy(rows_v, xs_hbm.at[p0_v], s0)
            c1 = pltpu.make_async_copy(rows_v, xs_hbm.at[p1_v], s1)
            c0.start()
            c1.start()
            c0.wait()
            c1.wait()

    x_sorted = dispatch(hs, pos0, pos1)

    nff = ff // FFB
    stage_a = pl.pallas_call(
        _stage_a_body,
        grid_spec=pltpu.PrefetchScalarGridSpec(
            num_scalar_prefetch=2,
            grid=(nff, nt),
            in_specs=[
                pl.BlockSpec((BLK, h),
                             lambda fj, i, te, na: (jnp.minimum(i, na[0] - 1), 0)),
                pl.BlockSpec((1, FFB, h),
                             lambda fj, i, te, na: (te[jnp.minimum(i, na[0] - 1)], fj, 0)),
                pl.BlockSpec((1, FFB, h),
                             lambda fj, i, te, na: (te[jnp.minimum(i, na[0] - 1)], fj, 0)),
            ],
            out_specs=pl.BlockSpec((BLK, FFB), lambda fj, i, te, na: (i, fj)),
        ),
        out_shape=jax.ShapeDtypeStruct((p, ff), jnp.float32),
        compiler_params=pltpu.CompilerParams(
            dimension_semantics=("arbitrary", "arbitrary")),
    )
    g = stage_a(te, na, x_sorted, w1, w3)

    stage_b = pl.pallas_call(
        _stage_b_body,
        grid_spec=pltpu.PrefetchScalarGridSpec(
            num_scalar_prefetch=2,
            grid=(h // HB, nt),
            in_specs=[
                pl.BlockSpec((BLK, ff),
                             lambda hb, i, te, na: (jnp.minimum(i, na[0] - 1), 0)),
                pl.BlockSpec((1, HB, ff),
                             lambda hb, i, te, na: (te[jnp.minimum(i, na[0] - 1)], hb, 0)),
            ],
            out_specs=pl.BlockSpec((BLK, HB), lambda hb, i, te, na: (i, hb)),
        ),
        out_shape=jax.ShapeDtypeStruct((p, h), jnp.float32),
        compiler_params=pltpu.CompilerParams(
            dimension_semantics=("arbitrary", "arbitrary")),
    )
    out_sorted = stage_b(te, na, g, w2)

    c2 = 16
    nchunk2 = tw // c2
    hv = h // 16

    @functools.partial(
        pl.kernel,
        out_type=jax.ShapeDtypeStruct((s, h), jnp.float32),
        mesh=plsc.VectorSubcoreMesh(core_axis_name="c", subcore_axis_name="s"),
        scratch_types=[
            pltpu.VMEM((c2, h), jnp.float32),
            pltpu.VMEM((c2, h), jnp.float32),
            pltpu.VMEM((c2, h), jnp.float32),
            pltpu.VMEM((c2,), jnp.int32),
            pltpu.VMEM((c2,), jnp.int32),
            pltpu.VMEM((c2,), jnp.float32),
            pltpu.VMEM((c2,), jnp.float32),
            pltpu.SemaphoreType.DMA,
            pltpu.SemaphoreType.DMA,
        ],
    )
    def combine(os_hbm, p0_hbm, p1_hbm, w0_hbm, w1_hbm, fin_hbm,
                a_v, b_v, o_v, p0_v, p1_v, w0_v, w1_v, s0, s1):
        wid = lax.axis_index("s") * nc + lax.axis_index("c")
        for k in range(nchunk2):
            base = wid * tw + k * c2
            pltpu.sync_copy(p0_hbm.at[pl.ds(base, c2)], p0_v)
            pltpu.sync_copy(p1_hbm.at[pl.ds(base, c2)], p1_v)
            pltpu.sync_copy(w0_hbm.at[pl.ds(base, c2)], w0_v)
            pltpu.sync_copy(w1_hbm.at[pl.ds(base, c2)], w1_v)
            ca = pltpu.make_async_copy(os_hbm.at[p0_v], a_v, s0)
            cb = pltpu.make_async_copy(os_hbm.at[p1_v], b_v, s1)
            ca.start()
            cb.start()
            ca.wait()
            cb.wait()
            w0vec = w0_v[...]
            w1vec = w1_v[...]
            gdn = lax.GatherDimensionNumbers(
                offset_dims=(), collapsed_slice_dims=(0,), start_index_map=(0,))
            for j in range(c2):
                jvec = jnp.full((c2, 1), j, jnp.int32)
                w0s = lax.gather(w0vec, jvec, gdn, (1,),
                                 mode=lax.GatherScatterMode.PROMISE_IN_BOUNDS)
                w1s = lax.gather(w1vec, jvec, gdn, (1,),
                                 mode=lax.GatherScatterMode.PROMISE_IN_BOUNDS)

                def hbody(hh, carry):
                    for u in range(4):
                        sl = pl.ds((hh * 4 + u) * 16, 16)
                        o_v[j, sl] = w0s * a_v[j, sl] + w1s * b_v[j, sl]
                    return carry

                lax.fori_loop(0, hv // 4, hbody, 0)
            pltpu.sync_copy(o_v, fin_hbm.at[pl.ds(base, c2)])

    final = combine(out_sorted, pos0, pos1, w0n, w1n)
    return final.reshape(b, s, h), logits
